# Initial kernel scaffold; baseline (speedup 1.0000x reference)
import jax
import jax.numpy as jnp
from jax.experimental import pallas as pl


def kernel(x, tok_emb_table, pos_emb_table):
    raise NotImplementedError("write your pallas kernel here")


# speedup vs baseline: 1.2669x; 1.2669x over previous
import functools

import jax
import jax.numpy as jnp
from jax import lax
from jax.experimental import pallas as pl
from jax.experimental.pallas import tpu as pltpu
from jax.experimental.pallas import tpu_sc as plsc

B, T, D = 4, 2048, 128
N = B * T
NC, NS = 2, 16
NW = NC * NS
R = N // NW
CH = 128
NCH = R // CH

mesh = plsc.VectorSubcoreMesh(core_axis_name="c", subcore_axis_name="s")


@functools.partial(
    pl.kernel,
    mesh=mesh,
    out_type=jax.ShapeDtypeStruct((N, D), jnp.float32),
    scratch_types=[
        pltpu.VMEM((NCH, CH), jnp.int32),
        pltpu.VMEM((R, D), jnp.float32),
        pltpu.VMEM((R, D), jnp.float32),
        pltpu.SemaphoreType.DMA,
        pltpu.SemaphoreType.DMA,
    ],
)
def _emb_kernel(x_hbm, tok_hbm, pos_hbm, out_hbm, idx_v, rows_v, pos_v,
                sem_g, sem_p):
    wid = lax.axis_index("s") * NC + lax.axis_index("c")
    base = wid * R
    pbase = lax.rem(base, T)

    pltpu.sync_copy(x_hbm.at[pl.ds(wid * NCH, NCH)], idx_v)

    cp_pos = pltpu.async_copy(pos_hbm.at[pl.ds(pbase, R)], pos_v, sem_p)

    cps = []
    for j in range(NCH):
        cps.append(
            pltpu.async_copy(
                tok_hbm.at[idx_v.at[j]],
                rows_v.at[pl.ds(j * CH, CH)],
                sem_g,
            )
        )
    for cp in cps:
        cp.wait()
    cp_pos.wait()

    def body(r, carry):
        for c in range(D // 16):
            sl = pl.ds(c * 16, 16)
            rows_v[r, sl] = rows_v[r, sl] + pos_v[r, sl]
        return carry

    lax.fori_loop(0, R, body, 0)

    pltpu.sync_copy(rows_v, out_hbm.at[pl.ds(base, R)])


def kernel(x, tok_emb_table, pos_emb_table):
    x2 = x.astype(jnp.int32).reshape(N // CH, CH)
    out = _emb_kernel(x2, tok_emb_table, pos_emb_table)
    return out.reshape(B, T, D)


# speedup vs baseline: 1.3527x; 1.0677x over previous
import functools

import jax
import jax.numpy as jnp
from jax import lax
from jax.experimental import pallas as pl
from jax.experimental.pallas import tpu as pltpu
from jax.experimental.pallas import tpu_sc as plsc

B, T, D = 4, 2048, 128
N = B * T
NC, NS = 2, 16
NW = NC * NS
TW = T // NW
LG = D // 16

mesh = plsc.VectorSubcoreMesh(core_axis_name="c", subcore_axis_name="s")


@functools.partial(
    pl.kernel,
    mesh=mesh,
    out_type=jax.ShapeDtypeStruct((N, D), jnp.float32),
    scratch_types=[
        pltpu.VMEM((B, TW), jnp.int32),
        pltpu.VMEM((B * TW, D), jnp.float32),
        pltpu.VMEM((TW, D), jnp.float32),
        pltpu.SemaphoreType.DMA,
        pltpu.SemaphoreType.DMA,
        pltpu.SemaphoreType.DMA,
        pltpu.SemaphoreType.DMA,
        pltpu.SemaphoreType.DMA,
        pltpu.SemaphoreType.DMA,
    ],
)
def _emb_kernel(xr_hbm, tok_hbm, pos_hbm, out_hbm, idx_v, rows_v, pos_v,
                sem_p, sem_g0, sem_g1, sem_g2, sem_g3, sem_w):
    sem_g = [sem_g0, sem_g1, sem_g2, sem_g3]
    wid = lax.axis_index("s") * NC + lax.axis_index("c")
    tbase = wid * TW

    pltpu.sync_copy(xr_hbm.at[wid], idx_v)

    cp_pos = pltpu.async_copy(pos_hbm.at[pl.ds(tbase, TW)], pos_v, sem_p)

    gcps = [
        pltpu.async_copy(
            tok_hbm.at[idx_v.at[b]],
            rows_v.at[pl.ds(b * TW, TW)],
            sem_g[b],
        )
        for b in range(B)
    ]
    cp_pos.wait()

    wcps = []
    for b in range(B):
        gcps[b].wait()

        def body(t, carry, b=b):
            r = b * TW + t
            for g in range(LG):
                sl = pl.ds(g * 16, 16)
                rows_v[r, sl] = rows_v[r, sl] + pos_v[t, sl]
            return carry

        lax.fori_loop(0, TW, body, 0)
        wcps.append(
            pltpu.async_copy(
                rows_v.at[pl.ds(b * TW, TW)],
                out_hbm.at[pl.ds(b * T + tbase, TW)],
                sem_w,
            )
        )
    for cp in wcps:
        cp.wait()


def kernel(x, tok_emb_table, pos_emb_table):
    xr = x.astype(jnp.int32).reshape(B, NW, TW).transpose(1, 0, 2)
    out = _emb_kernel(xr, tok_emb_table, pos_emb_table)
    return out.reshape(B, T, D)
